# baseline (speedup 1.0000x reference)
import jax
import jax.numpy as jnp
from jax.experimental import pallas as pl
from jax.experimental.pallas import tpu as pltpu

SPAN = 128
STRIDE = 4
LQ = 2048
HEADS = 12
DQK = 64
DV = 64
DIM = 768
KV_USED = (LQ - 1) // STRIDE + 1
SCALE = 1.0 / (DQK ** 0.5)

QT = 256
WIN = 128
NT = LQ // QT


def _dot_t(a, b):
    return jax.lax.dot_general(a, b, (((1,), (1,)), ((), ())),
                               preferred_element_type=jnp.float32)


def _mha_kernel(q_ref, k_ref, v_ref, wq_ref, wk_ref, wv_ref, wout_ref, out_ref,
                kf_s, vf_s):
    t = pl.program_id(0)

    @pl.when(t == 0)
    def _():
        kf_s[...] = _dot_t(k_ref[...], wk_ref[...])
        vf_s[...] = _dot_t(v_ref[...], wv_ref[...])

    qt = _dot_t(q_ref[...], wq_ref[...])

    r = jax.lax.broadcasted_iota(jnp.int32, (QT, WIN), 0)
    c4 = STRIDE * jax.lax.broadcasted_iota(jnp.int32, (QT, WIN), 1)
    shift = jnp.where(t == 0, 0, QT)
    d = r + shift - c4
    bias = jnp.where((d >= 0) & (d < SPAN), 0.0, -jnp.inf).astype(jnp.float32)

    lo = (WIN // 2) * jnp.maximum(t - 1, 0)
    kwin = kf_s[pl.ds(lo, WIN), :]
    vwin = vf_s[pl.ds(lo, WIN), :]
    ohs = []
    for h in range(HEADS):
        qh = qt[:, h * DQK:(h + 1) * DQK]
        kh = kwin[:, h * DQK:(h + 1) * DQK]
        vh = vwin[:, h * DV:(h + 1) * DV]
        s = _dot_t(qh, kh) * SCALE + bias
        m = jnp.max(s, axis=1, keepdims=True)
        e = jnp.exp(s - m)
        denom = jnp.sum(e, axis=1, keepdims=True)
        av = jax.lax.dot_general(e, vh, (((1,), (0,)), ((), ())),
                                 preferred_element_type=jnp.float32)
        ohs.append(av / denom)
    qkv = jnp.concatenate(ohs, axis=1).astype(jnp.bfloat16)
    out_ref[...] = _dot_t(qkv, wout_ref[...])


def kernel(q, k, v, Wq, Wk, Wv, Wout):
    batch = q.shape[0]
    bf16 = jnp.bfloat16
    q2 = q.reshape(batch * LQ, DIM).astype(bf16)
    k2 = k.reshape(-1, DIM).astype(bf16)
    v2 = v.reshape(-1, DIM).astype(bf16)
    Wq = Wq.astype(bf16)
    Wk = Wk.astype(bf16)
    Wv = Wv.astype(bf16)
    Wout = Wout.astype(bf16)
    out = pl.pallas_call(
        _mha_kernel,
        grid=(NT,),
        in_specs=[
            pl.BlockSpec((QT, DIM), lambda t: (t, 0)),
            pl.BlockSpec((KV_USED, DIM), lambda t: (0, 0)),
            pl.BlockSpec((KV_USED, DIM), lambda t: (0, 0)),
            pl.BlockSpec((HEADS * DQK, DIM), lambda t: (0, 0)),
            pl.BlockSpec((HEADS * DQK, DIM), lambda t: (0, 0)),
            pl.BlockSpec((HEADS * DV, DIM), lambda t: (0, 0)),
            pl.BlockSpec((DIM, HEADS * DV), lambda t: (0, 0)),
        ],
        out_specs=pl.BlockSpec((QT, DIM), lambda t: (t, 0)),
        out_shape=jax.ShapeDtypeStruct((LQ, DIM), jnp.float32),
        scratch_shapes=[
            pltpu.VMEM((KV_USED, HEADS * DQK), jnp.float32),
            pltpu.VMEM((KV_USED, HEADS * DV), jnp.float32),
        ],
    )(q2, k2, v2, Wq, Wk, Wv, Wout)
    return out.reshape(batch, LQ, DIM)
